# baseline (speedup 1.0000x reference)
import jax
import jax.numpy as jnp
from jax import lax
from jax.experimental import pallas as pl
from jax.experimental.pallas import tpu as pltpu
from jax.experimental.pallas import tpu_sc as plsc

B = 16384
D = 32
L = 16
NC, NS = 2, 16
NW = NC * NS
BPW = B // NW
CHUNK = 128
NCH = BPW // CHUNK

_DIMNUMS = lax.GatherDimensionNumbers(
    offset_dims=(), collapsed_slice_dims=(0,), start_index_map=(0,))


def _vtake(v, idx):
    return lax.gather(v, idx[:, None], _DIMNUMS, slice_sizes=(1,),
                      mode=lax.GatherScatterMode.PROMISE_IN_BOUNDS)


def _gmf_body(xf_hbm, w_hbm, uid_hbm, iid_hbm, out_hbm,
              xv, idxu, idxi, uv, iv, wv, outv, sem_u, sem_i):
    wid = lax.axis_index("s") * NC + lax.axis_index("c")
    base = wid * BPW

    pltpu.sync_copy(xf_hbm.at[pl.ds(base * 2, BPW * 2)], xv)
    pltpu.sync_copy(w_hbm, wv)

    lane = lax.iota(jnp.int32, L)
    even = (2 * lane) % L
    lo_half = lane < (L // 2)

    for c in range(BPW // L):
        f0 = xv[pl.ds(c * 2 * L, L)]
        f1 = xv[pl.ds(c * 2 * L + L, L)]
        u = jnp.where(lo_half, _vtake(f0, even), _vtake(f1, even))
        i = jnp.where(lo_half, _vtake(f0, even + 1), _vtake(f1, even + 1))
        j, off = divmod(c * L, CHUNK)
        idxu[j, pl.ds(off, L)] = u
        idxi[j, pl.ds(off, L)] = i

    cps = []
    for j in range(NCH):
        cps.append(pltpu.async_copy(
            uid_hbm.at[idxu.at[j]], uv.at[pl.ds(j * CHUNK, CHUNK)], sem_u))
        cps.append(pltpu.async_copy(
            iid_hbm.at[idxi.at[j]], iv.at[pl.ds(j * CHUNK, CHUNK)], sem_i))
    for cp in cps:
        cp.wait()

    w0 = wv[pl.ds(0, L)]
    w1 = wv[pl.ds(L, L)]

    rev = ((lane & 1) << 3) | ((lane & 2) << 1) | ((lane & 4) >> 1) | ((lane & 8) >> 3)

    def group(g, _):
        vecs = []
        for k in range(L):
            r = g * L + k
            vecs.append(uv[r, pl.ds(0, L)] * iv[r, pl.ds(0, L)] * w0
                        + uv[r, pl.ds(L, L)] * iv[r, pl.ds(L, L)] * w1)
        s = L // 2
        while len(vecs) > 1:
            pick_a = (lane & s) == 0
            vecs = [jnp.where(pick_a, a + _vtake(a, lane ^ s), b + _vtake(b, lane ^ s))
                    for a, b in zip(vecs[0::2], vecs[1::2])]
            s //= 2
        outv[pl.ds(g * L, L)] = _vtake(vecs[0], rev)
        return _
    lax.fori_loop(0, BPW // L, group, 0)

    pltpu.sync_copy(outv, out_hbm.at[pl.ds(base, BPW)])


@jax.jit
def _gmf(xf, w_flat, uid_table, iid_table):
    mesh = plsc.VectorSubcoreMesh(
        core_axis_name="c", subcore_axis_name="s", num_cores=NC, num_subcores=NS)
    return pl.kernel(
        _gmf_body,
        out_type=jax.ShapeDtypeStruct((B,), jnp.float32),
        mesh=mesh,
        compiler_params=pltpu.CompilerParams(use_tc_tiling_on_sc=False),
        scratch_types=[
            pltpu.VMEM((BPW * 2,), jnp.int32),
            pltpu.VMEM((NCH, CHUNK), jnp.int32),
            pltpu.VMEM((NCH, CHUNK), jnp.int32),
            pltpu.VMEM((BPW, D), jnp.float32),
            pltpu.VMEM((BPW, D), jnp.float32),
            pltpu.VMEM((D,), jnp.float32),
            pltpu.VMEM((BPW,), jnp.float32),
            pltpu.SemaphoreType.DMA,
            pltpu.SemaphoreType.DMA,
        ],
    )(xf, w_flat, uid_table, iid_table)


def kernel(x, uid_table, iid_table, W):
    return _gmf(x.reshape(B * 2), W.reshape(D), uid_table, iid_table)


# speedup vs baseline: 1.3538x; 1.3538x over previous
import jax
import jax.numpy as jnp
from jax import lax
from jax.experimental import pallas as pl
from jax.experimental.pallas import tpu as pltpu
from jax.experimental.pallas import tpu_sc as plsc

B = 16384
D = 32
L = 16
NC, NS = 2, 16
NW = NC * NS
BPW = B // NW
WAVE = 32
NWAVE = BPW // WAVE
CPW = WAVE // L

_DIMNUMS = lax.GatherDimensionNumbers(
    offset_dims=(), collapsed_slice_dims=(0,), start_index_map=(0,))


def _vtake(v, idx):
    return lax.gather(v, idx[:, None], _DIMNUMS, slice_sizes=(1,),
                      mode=lax.GatherScatterMode.PROMISE_IN_BOUNDS)


def _gmf_body(xf_hbm, w_hbm, uid_hbm, iid_hbm, out_hbm,
              xv, u8, i8, wv, outv, sem_u, sem_i):
    wid = lax.axis_index("s") * NC + lax.axis_index("c")
    base = wid * BPW

    pltpu.sync_copy(xf_hbm.at[pl.ds(base * 2, BPW * 2)], xv)
    pltpu.sync_copy(w_hbm, wv)

    lane = lax.iota(jnp.int32, L)
    even = (2 * lane) % L
    lo_half = lane < (L // 2)
    w0 = wv[pl.ds(0, L)]
    w1 = wv[pl.ds(L, L)]
    rev = ((lane & 1) << 3) | ((lane & 2) << 1) | ((lane & 4) >> 1) | ((lane & 8) >> 3)

    def wave(wi, _):
        cps = []
        subs_u = []
        subs_i = []
        for c in range(CPW):
            off = wi * 2 * WAVE + c * 2 * L
            f0 = xv[pl.ds(off, L)]
            f1 = xv[pl.ds(off + L, L)]
            u_ids = jnp.where(lo_half, _vtake(f0, even), _vtake(f1, even))
            i_ids = jnp.where(lo_half, _vtake(f0, even + 1), _vtake(f1, even + 1))
            subs_u.append(u_ids & 7)
            subs_i.append(i_ids & 7)
            gu = u_ids - (u_ids & 7)
            gi = i_ids - (i_ids & 7)
            for k in range(L):
                slot = c * L + k
                su = pl.multiple_of(gu[k], 8)
                si = pl.multiple_of(gi[k], 8)
                cps.append(pltpu.async_copy(
                    uid_hbm.at[pl.ds(su, 8), :], u8.at[slot], sem_u))
                cps.append(pltpu.async_copy(
                    iid_hbm.at[pl.ds(si, 8), :], i8.at[slot], sem_i))
        for cp in cps:
            cp.wait()

        for c in range(CPW):
            vecs = []
            for k in range(L):
                slot = c * L + k
                su = subs_u[c][k]
                si = subs_i[c][k]
                vecs.append(
                    u8[slot, su, pl.ds(0, L)] * i8[slot, si, pl.ds(0, L)] * w0
                    + u8[slot, su, pl.ds(L, L)] * i8[slot, si, pl.ds(L, L)] * w1)
            s = L // 2
            while len(vecs) > 1:
                pick_a = (lane & s) == 0
                vecs = [jnp.where(pick_a, a + _vtake(a, lane ^ s),
                                  b + _vtake(b, lane ^ s))
                        for a, b in zip(vecs[0::2], vecs[1::2])]
                s //= 2
            outv[pl.ds(wi * WAVE + c * L, L)] = _vtake(vecs[0], rev)
        return _
    lax.fori_loop(0, NWAVE, wave, 0)

    pltpu.sync_copy(outv, out_hbm.at[pl.ds(base, BPW)])


@jax.jit
def _gmf(xf, w_flat, uid_table, iid_table):
    mesh = plsc.VectorSubcoreMesh(
        core_axis_name="c", subcore_axis_name="s", num_cores=NC, num_subcores=NS)
    return pl.kernel(
        _gmf_body,
        out_type=jax.ShapeDtypeStruct((B,), jnp.float32),
        mesh=mesh,
        scratch_types=[
            pltpu.VMEM((BPW * 2,), jnp.int32),
            pltpu.VMEM((WAVE, 8, D), jnp.float32),
            pltpu.VMEM((WAVE, 8, D), jnp.float32),
            pltpu.VMEM((D,), jnp.float32),
            pltpu.VMEM((BPW,), jnp.float32),
            pltpu.SemaphoreType.DMA,
            pltpu.SemaphoreType.DMA,
        ],
    )(xf, w_flat, uid_table, iid_table)


def kernel(x, uid_table, iid_table, W):
    return _gmf(x.reshape(B * 2), W.reshape(D), uid_table, iid_table)
